# Initial kernel scaffold; baseline (speedup 1.0000x reference)
import jax
import jax.numpy as jnp
from jax.experimental import pallas as pl


def kernel(x, Wqk, bqk, Wv, bv, Wo, bo, rot):
    raise NotImplementedError("write your pallas kernel here")


# speedup vs baseline: 6.9804x; 6.9804x over previous
import functools
import math

import jax
import jax.numpy as jnp
from jax.experimental import pallas as pl


def _qkv_kernel(x_ref, w_ref, b_ref, out_ref):
    acc = jnp.dot(x_ref[...], w_ref[...], preferred_element_type=jnp.float32)
    out_ref[...] = acc + b_ref[...]


def _attn_kernel(q_ref, k_ref, v_ref, wo_ref, bo_ref, out_ref):
    h = pl.program_id(1)
    q = q_ref[0] * (1.0 / math.sqrt(q_ref.shape[-1]))
    k = k_ref[0]
    v = v_ref[0]
    dots = jax.lax.dot_general(
        q, k, (((1,), (1,)), ((), ())), preferred_element_type=jnp.float32
    )
    m = jnp.max(dots, axis=-1, keepdims=True)
    e = jnp.exp(dots - m)
    attn = e / jnp.sum(e, axis=-1, keepdims=True)
    o = jnp.dot(attn, v, preferred_element_type=jnp.float32)
    contrib = jnp.dot(o, wo_ref[0], preferred_element_type=jnp.float32)

    @pl.when(h == 0)
    def _():
        out_ref[...] = contrib + bo_ref[...]

    @pl.when(h != 0)
    def _():
        out_ref[...] += contrib


@functools.partial(jax.jit, static_argnames=())
def kernel(x, Wqk, bqk, Wv, bv, Wo, bo, rot):
    del rot
    B, L, D = x.shape
    H = 16
    Dh = D // H
    x2 = x.reshape(L, D)

    Wall_T = jnp.concatenate([Wqk, Wv], axis=0).T
    ball = jnp.concatenate([bqk, bv]).reshape(1, 3 * D)
    C = 3 * D
    CB = C // 4
    qkv = pl.pallas_call(
        _qkv_kernel,
        grid=(C // CB,),
        in_specs=[
            pl.BlockSpec((L, D), lambda j: (0, 0)),
            pl.BlockSpec((D, CB), lambda j: (0, j)),
            pl.BlockSpec((1, CB), lambda j: (0, j)),
        ],
        out_specs=pl.BlockSpec((L, CB), lambda j: (0, j)),
        out_shape=jax.ShapeDtypeStruct((L, C), jnp.float32),
    )(x2, Wall_T, ball)

    qkv_h = qkv.reshape(L, 3 * H, Dh).transpose(1, 0, 2)
    WoT_h = Wo.T.reshape(H, Dh, D)
    bo2 = bo.reshape(1, D)
    QB = L // 2
    out = pl.pallas_call(
        _attn_kernel,
        grid=(L // QB, H),
        in_specs=[
            pl.BlockSpec((1, QB, Dh), lambda qb, h: (h, qb, 0)),
            pl.BlockSpec((1, L, Dh), lambda qb, h: (H + h, 0, 0)),
            pl.BlockSpec((1, L, Dh), lambda qb, h: (2 * H + h, 0, 0)),
            pl.BlockSpec((1, Dh, D), lambda qb, h: (h, 0, 0)),
            pl.BlockSpec((1, D), lambda qb, h: (0, 0)),
        ],
        out_specs=pl.BlockSpec((QB, D), lambda qb, h: (qb, 0)),
        out_shape=jax.ShapeDtypeStruct((L, D), jnp.float32),
    )(qkv_h, qkv_h, qkv_h, WoT_h, bo2)

    return out.reshape(B, L, D)


# speedup vs baseline: 9.4474x; 1.3534x over previous
import functools
import math

import jax
import jax.numpy as jnp
from jax.experimental import pallas as pl


def _qkv_kernel(x_ref, w_ref, b_ref, out_ref):
    acc = jnp.dot(x_ref[...], w_ref[...], preferred_element_type=jnp.float32)
    out_ref[...] = acc + b_ref[...]


def _attn_kernel(q_ref, k_ref, v_ref, wo_ref, bo_ref, out_ref):
    hp = pl.program_id(1)
    Dh = 64
    scale = 1.0 / math.sqrt(Dh)
    contrib = None
    for i in range(2):
        q = q_ref[:, i * Dh:(i + 1) * Dh] * scale
        k = k_ref[:, i * Dh:(i + 1) * Dh]
        v = v_ref[:, i * Dh:(i + 1) * Dh]
        dots = jax.lax.dot_general(
            q, k, (((1,), (1,)), ((), ())), preferred_element_type=jnp.float32
        )
        m = jnp.max(dots, axis=-1, keepdims=True)
        e = jnp.exp(dots - m)
        attn = e / jnp.sum(e, axis=-1, keepdims=True)
        o = jnp.dot(attn, v, preferred_element_type=jnp.float32)
        c = jnp.dot(o, wo_ref[0, i * Dh:(i + 1) * Dh, :],
                    preferred_element_type=jnp.float32)
        contrib = c if contrib is None else contrib + c

    @pl.when(hp == 0)
    def _():
        out_ref[...] = contrib + bo_ref[...]

    @pl.when(hp != 0)
    def _():
        out_ref[...] += contrib


@functools.partial(jax.jit, static_argnames=())
def kernel(x, Wqk, bqk, Wv, bv, Wo, bo, rot):
    del rot
    B, L, D = x.shape
    H = 16
    Dh = D // H
    x2 = x.reshape(L, D)

    Wall_T = jnp.concatenate([Wqk, Wv], axis=0).T
    ball = jnp.concatenate([bqk, bv]).reshape(1, 3 * D)
    C = 3 * D
    CB = C // 4
    qkv = pl.pallas_call(
        _qkv_kernel,
        grid=(C // CB,),
        in_specs=[
            pl.BlockSpec((L, D), lambda j: (0, 0)),
            pl.BlockSpec((D, CB), lambda j: (0, j)),
            pl.BlockSpec((1, CB), lambda j: (0, j)),
        ],
        out_specs=pl.BlockSpec((L, CB), lambda j: (0, j)),
        out_shape=jax.ShapeDtypeStruct((L, C), jnp.float32),
    )(x2, Wall_T, ball)

    HP = H // 2
    WoT_h = Wo.T.reshape(HP, 2 * Dh, D)
    bo2 = bo.reshape(1, D)
    QB = L // 2
    out = pl.pallas_call(
        _attn_kernel,
        grid=(L // QB, HP),
        in_specs=[
            pl.BlockSpec((QB, 2 * Dh), lambda qb, hp: (qb, hp)),
            pl.BlockSpec((L, 2 * Dh), lambda qb, hp: (0, HP + hp)),
            pl.BlockSpec((L, 2 * Dh), lambda qb, hp: (0, 2 * HP + hp)),
            pl.BlockSpec((1, 2 * Dh, D), lambda qb, hp: (hp, 0, 0)),
            pl.BlockSpec((1, D), lambda qb, hp: (0, 0)),
        ],
        out_specs=pl.BlockSpec((QB, D), lambda qb, hp: (qb, 0)),
        out_shape=jax.ShapeDtypeStruct((L, D), jnp.float32),
    )(qkv, qkv, qkv, WoT_h, bo2)

    return out.reshape(B, L, D)


# speedup vs baseline: 9.4909x; 1.0046x over previous
import functools
import math

import jax
import jax.numpy as jnp
from jax.experimental import pallas as pl


def _qkv_kernel(x_ref, w_ref, b_ref, out_ref):
    acc = jnp.dot(x_ref[...], w_ref[...], preferred_element_type=jnp.float32)
    out_ref[...] = (acc + b_ref[...]).astype(out_ref.dtype)


def _attn_kernel(q_ref, k_ref, v_ref, wo_ref, bo_ref, out_ref):
    hp = pl.program_id(1)
    Dh = 64
    scale = 1.0 / math.sqrt(Dh)
    contrib = None
    for i in range(2):
        q = q_ref[:, i * Dh:(i + 1) * Dh]
        k = k_ref[:, i * Dh:(i + 1) * Dh]
        v = v_ref[:, i * Dh:(i + 1) * Dh]
        dots = jax.lax.dot_general(
            q, k, (((1,), (1,)), ((), ())), preferred_element_type=jnp.float32
        ) * scale
        m = jnp.max(dots, axis=-1, keepdims=True)
        e = jnp.exp(dots - m)
        attn = (e / jnp.sum(e, axis=-1, keepdims=True)).astype(jnp.bfloat16)
        o = jnp.dot(attn, v, preferred_element_type=jnp.float32)
        c = jnp.dot(o.astype(jnp.bfloat16), wo_ref[0, i * Dh:(i + 1) * Dh, :],
                    preferred_element_type=jnp.float32)
        contrib = c if contrib is None else contrib + c

    @pl.when(hp == 0)
    def _():
        out_ref[...] = contrib + bo_ref[...]

    @pl.when(hp != 0)
    def _():
        out_ref[...] += contrib


@functools.partial(jax.jit, static_argnames=())
def kernel(x, Wqk, bqk, Wv, bv, Wo, bo, rot):
    del rot
    B, L, D = x.shape
    H = 16
    Dh = D // H
    x2 = x.reshape(L, D)

    Wall_T = jnp.concatenate([Wqk, Wv], axis=0).T.astype(jnp.bfloat16)
    ball = jnp.concatenate([bqk, bv]).reshape(1, 3 * D)
    x2b = x2.astype(jnp.bfloat16)
    C = 3 * D
    CB = C // 4
    qkv = pl.pallas_call(
        _qkv_kernel,
        grid=(C // CB,),
        in_specs=[
            pl.BlockSpec((L, D), lambda j: (0, 0)),
            pl.BlockSpec((D, CB), lambda j: (0, j)),
            pl.BlockSpec((1, CB), lambda j: (0, j)),
        ],
        out_specs=pl.BlockSpec((L, CB), lambda j: (0, j)),
        out_shape=jax.ShapeDtypeStruct((L, C), jnp.bfloat16),
    )(x2b, Wall_T, ball)

    HP = H // 2
    WoT_h = Wo.T.reshape(HP, 2 * Dh, D).astype(jnp.bfloat16)
    bo2 = bo.reshape(1, D)
    QB = L // 2
    out = pl.pallas_call(
        _attn_kernel,
        grid=(L // QB, HP),
        in_specs=[
            pl.BlockSpec((QB, 2 * Dh), lambda qb, hp: (qb, hp)),
            pl.BlockSpec((L, 2 * Dh), lambda qb, hp: (0, HP + hp)),
            pl.BlockSpec((L, 2 * Dh), lambda qb, hp: (0, 2 * HP + hp)),
            pl.BlockSpec((1, 2 * Dh, D), lambda qb, hp: (hp, 0, 0)),
            pl.BlockSpec((1, D), lambda qb, hp: (0, 0)),
        ],
        out_specs=pl.BlockSpec((QB, D), lambda qb, hp: (qb, 0)),
        out_shape=jax.ShapeDtypeStruct((L, D), jnp.float32),
    )(qkv, qkv, qkv, WoT_h, bo2)

    return out.reshape(B, L, D)


# speedup vs baseline: 10.4348x; 1.0994x over previous
import functools
import math

import jax
import jax.numpy as jnp
from jax.experimental import pallas as pl


def _qkv_kernel(x_ref, w_ref, b_ref, out_ref):
    acc = jnp.dot(x_ref[...], w_ref[...], preferred_element_type=jnp.float32)
    out_ref[...] = (acc + b_ref[...]).astype(out_ref.dtype)


def _attn_kernel(q_ref, k_ref, v_ref, wo_ref, bo_ref, out_ref):
    hp = pl.program_id(1)
    Dh = 64
    contrib = None
    for i in range(2):
        q = q_ref[:, i * Dh:(i + 1) * Dh]
        k = k_ref[:, i * Dh:(i + 1) * Dh]
        v = v_ref[:, i * Dh:(i + 1) * Dh]
        dots = jax.lax.dot_general(
            q, k, (((1,), (1,)), ((), ())), preferred_element_type=jnp.float32
        )
        m = jnp.max(dots, axis=-1, keepdims=True)
        e = jnp.exp2(dots - m).astype(jnp.bfloat16)
        s = jnp.sum(e, axis=-1, keepdims=True, dtype=jnp.float32)
        o = jnp.dot(e, v, preferred_element_type=jnp.float32) / s
        c = jnp.dot(o.astype(jnp.bfloat16), wo_ref[0, i * Dh:(i + 1) * Dh, :],
                    preferred_element_type=jnp.float32)
        contrib = c if contrib is None else contrib + c

    @pl.when(hp == 0)
    def _():
        out_ref[...] = contrib + bo_ref[...]

    @pl.when(hp != 0)
    def _():
        out_ref[...] += contrib


@functools.partial(jax.jit, static_argnames=())
def kernel(x, Wqk, bqk, Wv, bv, Wo, bo, rot):
    del rot
    B, L, D = x.shape
    H = 16
    Dh = D // H
    x2 = x.reshape(L, D)

    qscale = math.log2(math.e) / math.sqrt(Dh)
    Wqk_s = Wqk.at[:D].multiply(qscale)
    bqk_s = bqk.at[:D].multiply(qscale)
    Wall_T = jnp.concatenate([Wqk_s, Wv], axis=0).T.astype(jnp.bfloat16)
    ball = jnp.concatenate([bqk_s, bv]).reshape(1, 3 * D)
    x2b = x2.astype(jnp.bfloat16)
    C = 3 * D
    CB = C // 4
    qkv = pl.pallas_call(
        _qkv_kernel,
        grid=(C // CB,),
        in_specs=[
            pl.BlockSpec((L, D), lambda j: (0, 0)),
            pl.BlockSpec((D, CB), lambda j: (0, j)),
            pl.BlockSpec((1, CB), lambda j: (0, j)),
        ],
        out_specs=pl.BlockSpec((L, CB), lambda j: (0, j)),
        out_shape=jax.ShapeDtypeStruct((L, C), jnp.bfloat16),
    )(x2b, Wall_T, ball)

    HP = H // 2
    WoT_h = Wo.T.reshape(HP, 2 * Dh, D).astype(jnp.bfloat16)
    bo2 = bo.reshape(1, D)
    QB = L // 2
    out = pl.pallas_call(
        _attn_kernel,
        grid=(L // QB, HP),
        in_specs=[
            pl.BlockSpec((QB, 2 * Dh), lambda qb, hp: (qb, hp)),
            pl.BlockSpec((L, 2 * Dh), lambda qb, hp: (0, HP + hp)),
            pl.BlockSpec((L, 2 * Dh), lambda qb, hp: (0, 2 * HP + hp)),
            pl.BlockSpec((1, 2 * Dh, D), lambda qb, hp: (hp, 0, 0)),
            pl.BlockSpec((1, D), lambda qb, hp: (0, 0)),
        ],
        out_specs=pl.BlockSpec((QB, D), lambda qb, hp: (qb, 0)),
        out_shape=jax.ShapeDtypeStruct((L, D), jnp.float32),
    )(qkv, qkv, qkv, WoT_h, bo2)

    return out.reshape(B, L, D)


# speedup vs baseline: 12.3155x; 1.1802x over previous
import functools
import math

import jax
import jax.numpy as jnp
from jax.experimental import pallas as pl


_QSCALE = math.log2(math.e) / 8.0


def _qkv_kernel(x_ref, wqk_ref, wv_ref, b_ref, out_ref):
    j = pl.program_id(0)
    xb = x_ref[...].astype(jnp.bfloat16)

    def proj(w):
        acc = jax.lax.dot_general(
            xb, w, (((1,), (1,)), ((), ())), preferred_element_type=jnp.float32
        )
        out_ref[...] = (acc + b_ref[0]).astype(jnp.bfloat16)

    @pl.when(j == 0)
    def _():
        proj((wqk_ref[...] * _QSCALE).astype(jnp.bfloat16))

    @pl.when(j == 1)
    def _():
        proj(wqk_ref[...].astype(jnp.bfloat16))

    @pl.when(j == 2)
    def _():
        proj(wv_ref[...].astype(jnp.bfloat16))


def _attn_kernel(q_ref, k_ref, v_ref, wo_ref, bo_ref, out_ref):
    hp = pl.program_id(1)
    Dh = 64
    wo = wo_ref[...].astype(jnp.bfloat16)
    contrib = None
    for i in range(2):
        q = q_ref[:, i * Dh:(i + 1) * Dh]
        k = k_ref[:, i * Dh:(i + 1) * Dh]
        v = v_ref[:, i * Dh:(i + 1) * Dh]
        dots = jax.lax.dot_general(
            q, k, (((1,), (1,)), ((), ())), preferred_element_type=jnp.float32
        )
        m = jnp.max(dots, axis=-1, keepdims=True)
        e = jnp.exp2(dots - m).astype(jnp.bfloat16)
        s = jnp.sum(e, axis=-1, keepdims=True, dtype=jnp.float32)
        o = jnp.dot(e, v, preferred_element_type=jnp.float32) / s
        c = jax.lax.dot_general(
            o.astype(jnp.bfloat16), wo[:, i * Dh:(i + 1) * Dh],
            (((1,), (1,)), ((), ())), preferred_element_type=jnp.float32,
        )
        contrib = c if contrib is None else contrib + c

    @pl.when(hp == 0)
    def _():
        out_ref[...] = contrib + bo_ref[...]

    @pl.when(hp != 0)
    def _():
        out_ref[...] += contrib


@functools.partial(jax.jit, static_argnames=())
def kernel(x, Wqk, bqk, Wv, bv, Wo, bo, rot):
    del rot
    B, L, D = x.shape
    H = 16
    Dh = D // H
    x2 = x.reshape(L, D)

    ball = jnp.concatenate([bqk.at[:D].multiply(_QSCALE), bv]).reshape(3, 1, D)
    qkv = pl.pallas_call(
        _qkv_kernel,
        grid=(3,),
        in_specs=[
            pl.BlockSpec((L, D), lambda j: (0, 0)),
            pl.BlockSpec((D, D), lambda j: (jnp.minimum(j, 1), 0)),
            pl.BlockSpec((D, D), lambda j: (0, 0)),
            pl.BlockSpec((1, 1, D), lambda j: (j, 0, 0)),
        ],
        out_specs=pl.BlockSpec((L, D), lambda j: (0, j)),
        out_shape=jax.ShapeDtypeStruct((L, 3 * D), jnp.bfloat16),
    )(x2, Wqk, Wv, ball)

    HP = H // 2
    bo2 = bo.reshape(1, D)
    QB = L // 2
    out = pl.pallas_call(
        _attn_kernel,
        grid=(L // QB, HP),
        in_specs=[
            pl.BlockSpec((QB, 2 * Dh), lambda qb, hp: (qb, hp)),
            pl.BlockSpec((L, 2 * Dh), lambda qb, hp: (0, HP + hp)),
            pl.BlockSpec((L, 2 * Dh), lambda qb, hp: (0, 2 * HP + hp)),
            pl.BlockSpec((D, 2 * Dh), lambda qb, hp: (0, hp)),
            pl.BlockSpec((1, D), lambda qb, hp: (0, 0)),
        ],
        out_specs=pl.BlockSpec((QB, D), lambda qb, hp: (qb, 0)),
        out_shape=jax.ShapeDtypeStruct((L, D), jnp.float32),
    )(qkv, qkv, qkv, Wo, bo2)

    return out.reshape(B, L, D)


# speedup vs baseline: 12.4140x; 1.0080x over previous
import functools
import math

import jax
import jax.numpy as jnp
from jax.experimental import pallas as pl


_QSCALE = math.log2(math.e) / 8.0


def _qkv_kernel(x_ref, wqk_ref, wv_ref, b_ref, out_ref):
    j = pl.program_id(0)
    xb = x_ref[...].astype(jnp.bfloat16)

    def proj(w):
        acc = jax.lax.dot_general(
            xb, w, (((1,), (1,)), ((), ())), preferred_element_type=jnp.float32
        )
        out_ref[...] = (acc + b_ref[0]).astype(jnp.bfloat16)

    @pl.when(j == 0)
    def _():
        proj((wqk_ref[...] * _QSCALE).astype(jnp.bfloat16))

    @pl.when(j == 1)
    def _():
        proj(wqk_ref[...].astype(jnp.bfloat16))

    @pl.when(j == 2)
    def _():
        proj(wv_ref[...].astype(jnp.bfloat16))


def _attn_kernel(q_ref, k_ref, v_ref, wo_ref, bo_ref, out_ref):
    hp = pl.program_id(1)
    Dh = 64
    wo = wo_ref[...].astype(jnp.bfloat16)
    contrib = None
    for i in range(2):
        q = q_ref[:, i * Dh:(i + 1) * Dh]
        k = k_ref[:, i * Dh:(i + 1) * Dh]
        v = v_ref[:, i * Dh:(i + 1) * Dh]
        dots = jax.lax.dot_general(
            q, k, (((1,), (1,)), ((), ())), preferred_element_type=jnp.float32
        )
        m = jnp.max(dots, axis=-1, keepdims=True)
        e = jnp.exp2(dots - m).astype(jnp.bfloat16)
        s = jnp.sum(e, axis=-1, keepdims=True, dtype=jnp.float32)
        o = jnp.dot(e, v, preferred_element_type=jnp.float32) / s
        c = jax.lax.dot_general(
            o.astype(jnp.bfloat16), wo[:, i * Dh:(i + 1) * Dh],
            (((1,), (1,)), ((), ())), preferred_element_type=jnp.float32,
        )
        contrib = c if contrib is None else contrib + c

    @pl.when(hp == 0)
    def _():
        out_ref[...] = contrib + bo_ref[...]

    @pl.when(hp != 0)
    def _():
        out_ref[...] += contrib


@functools.partial(jax.jit, static_argnames=())
def kernel(x, Wqk, bqk, Wv, bv, Wo, bo, rot):
    del rot
    B, L, D = x.shape
    H = 16
    Dh = D // H
    x2 = x.reshape(L, D)

    ball = jnp.concatenate([bqk.at[:D].multiply(_QSCALE), bv]).reshape(3, 1, D)
    qkv = pl.pallas_call(
        _qkv_kernel,
        grid=(3,),
        in_specs=[
            pl.BlockSpec((L, D), lambda j: (0, 0)),
            pl.BlockSpec((D, D), lambda j: (jnp.minimum(j, 1), 0)),
            pl.BlockSpec((D, D), lambda j: (0, 0)),
            pl.BlockSpec((1, 1, D), lambda j: (j, 0, 0)),
        ],
        out_specs=pl.BlockSpec((L, D), lambda j: (0, j)),
        out_shape=jax.ShapeDtypeStruct((L, 3 * D), jnp.bfloat16),
    )(x2, Wqk, Wv, ball)

    HP = H // 2
    bo2 = bo.reshape(1, D)
    QB = L
    out = pl.pallas_call(
        _attn_kernel,
        grid=(L // QB, HP),
        in_specs=[
            pl.BlockSpec((QB, 2 * Dh), lambda qb, hp: (qb, hp)),
            pl.BlockSpec((L, 2 * Dh), lambda qb, hp: (0, HP + hp)),
            pl.BlockSpec((L, 2 * Dh), lambda qb, hp: (0, 2 * HP + hp)),
            pl.BlockSpec((D, 2 * Dh), lambda qb, hp: (0, hp)),
            pl.BlockSpec((1, D), lambda qb, hp: (0, 0)),
        ],
        out_specs=pl.BlockSpec((QB, D), lambda qb, hp: (qb, 0)),
        out_shape=jax.ShapeDtypeStruct((L, D), jnp.float32),
    )(qkv, qkv, qkv, Wo, bo2)

    return out.reshape(B, L, D)
